# baseline (speedup 1.0000x reference)
import functools

import jax
import jax.numpy as jnp
from jax import lax
from jax.experimental import pallas as pl
from jax.experimental.pallas import tpu as pltpu
from jax.experimental.pallas import tpu_sc as plsc

N_CELLS = 10000
N_GENES = 128
E_CELL = 320000
E_GENE = 2048

_NC = 2
_NS = 16
_NW = _NC * _NS
_CK = 128
_CH = 80
_CHB = 88
_PH = 40
_PHS = 48
_EPW = _CHB * _CK
_EPAD = _NW * _EPW
_NPAD = 10112
_RPS = _NPAD // _NS
_XPAD = 10112
_XPS = _XPAD // _NS
_FH = 64

@functools.lru_cache(maxsize=1)
def _make_sc_cell_agg():
    mesh = plsc.VectorSubcoreMesh(core_axis_name="c", subcore_axis_name="s")

    @functools.partial(
        pl.kernel,
        mesh=mesh,
        out_type=jax.ShapeDtypeStruct((_NC, _NPAD, 128), jnp.float32),
        scratch_types=[
            pltpu.VMEM((_CHB, _CK), jnp.int32),
            pltpu.VMEM((_CHB, _CK), jnp.int32),
            pltpu.VMEM((_CK, 128), jnp.float32),
            pltpu.VMEM_SHARED((_NPAD, 128), jnp.float32),
            pltpu.SemaphoreType.DMA,
        ],
    )
    def _sc_cell_agg(xe_hbm, srcp_hbm, dstp_hbm, zeros_hbm, sum_out,
                     src_v, dst_v, buf_a, ssum, sem_a):
        cid = lax.axis_index("c")
        sid = lax.axis_index("s")
        w = cid * _NS + sid
        sl = pl.ds(sid * _RPS, _RPS)
        with jax.named_scope("agg_stage"):
            pltpu.sync_copy(zeros_hbm.at[sl], ssum.at[sl])
            pltpu.sync_copy(srcp_hbm.at[w], src_v)
            pltpu.sync_copy(dstp_hbm.at[w], dst_v)
        with jax.named_scope("agg_bar1"):
            plsc.subcore_barrier()

        def chunk(ci, carry):
            pltpu.async_copy(xe_hbm.at[src_v.at[ci]], buf_a, sem_a).wait()
            pltpu.sync_copy(buf_a, ssum.at[dst_v.at[ci]], add=True)
            return carry

        with jax.named_scope("agg_chunks"):
            lax.fori_loop(0, _CH, chunk, 0)
        with jax.named_scope("agg_bar2"):
            plsc.subcore_barrier()
        with jax.named_scope("agg_writeout"):
            pltpu.sync_copy(ssum.at[sl], sum_out.at[cid].at[sl])

    return _sc_cell_agg


_CNT_BLK = 12800


def _cnt_hist_body(dst_ref, o_ref):
    i = pl.program_id(0)

    @pl.when(i == 0)
    def _():
        o_ref[...] = jnp.zeros((80, 128), jnp.float32)

    d = dst_ref[...]
    hi = d >> 7
    lo = d & 127
    ioh = lax.broadcasted_iota(jnp.int32, (80, _CNT_BLK), 0)
    iol = lax.broadcasted_iota(jnp.int32, (128, _CNT_BLK), 0)
    dh = (ioh == hi).astype(jnp.float32)
    dl = (iol == lo).astype(jnp.float32)
    o_ref[...] += lax.dot_general(dh, dl, (((1,), (1,)), ((), ())),
                                  preferred_element_type=jnp.float32)


def _tc_cnt_hist(dst2):
    return pl.pallas_call(
        _cnt_hist_body,
        grid=(E_CELL // _CNT_BLK,),
        in_specs=[pl.BlockSpec((1, _CNT_BLK), lambda i: (0, i))],
        out_specs=pl.BlockSpec((80, 128), lambda i: (0, 0)),
        out_shape=jax.ShapeDtypeStruct((80, 128), jnp.float32),
    )(dst2)
def _cell_update_body(sum_ref, cnt_ref, x_ref, wl_ref, wr_ref, b_ref, o_ref):
    s = sum_ref[0] + sum_ref[1]
    r = 1.0 / jnp.clip(cnt_ref[...], 1.0, None)
    agg = s * r
    cdims = (((1,), (1,)), ((), ()))
    o_ref[...] = (
        lax.dot_general(agg, wl_ref[...], cdims,
                        preferred_element_type=jnp.float32)
        + lax.dot_general(x_ref[...], wr_ref[...], cdims,
                          preferred_element_type=jnp.float32)
        + b_ref[...]
    )


def _tc_cell_update(sum_parts, cnt_col, x, wl, wr, b2d):
    br = 1000
    grid = (N_CELLS // br,)
    return pl.pallas_call(
        _cell_update_body,
        grid=grid,
        in_specs=[
            pl.BlockSpec((_NC, br, 128), lambda i: (0, i, 0)),
            pl.BlockSpec((br, 1), lambda i: (i, 0)),
            pl.BlockSpec((br, 128), lambda i: (i, 0)),
            pl.BlockSpec((128, 128), lambda i: (0, 0)),
            pl.BlockSpec((128, 128), lambda i: (0, 0)),
            pl.BlockSpec((1, 128), lambda i: (0, 0)),
        ],
        out_specs=pl.BlockSpec((br, 128), lambda i: (i, 0)),
        out_shape=jax.ShapeDtypeStruct((N_CELLS, 128), jnp.float32),
    )(sum_parts, cnt_col, x, wl, wr, b2d)


def _gene_m_body(gg_ref, o_ref):
    gg = gg_ref[...]
    src = gg[0:1, :]
    dst = gg[1:2, :]
    ids = lax.broadcasted_iota(jnp.int32, (N_GENES, E_GENE), 0)
    dhot = (ids == dst).astype(jnp.float32)
    shot = (ids == src).astype(jnp.float32)
    m_raw = lax.dot_general(dhot, shot, (((1,), (1,)), ((), ())),
                            preferred_element_type=jnp.float32)
    cnt = jnp.sum(dhot, axis=1, keepdims=True)
    o_ref[...] = m_raw * (1.0 / jnp.clip(cnt, 1.0, None))


def _tc_gene_m(gene_graph):
    return pl.pallas_call(
        _gene_m_body,
        out_shape=jax.ShapeDtypeStruct((N_GENES, N_GENES), jnp.float32),
    )(gene_graph)


_GBC = 256
_GMAIN = (N_CELLS // _GBC) * _GBC
_GTAIL = N_CELLS - _GMAIN


def _gene_layer_body(xt_ref, m_ref, wl_ref, wr_ref, b_ref, o_ref, agg_ref):
    c = pl.program_id(0)

    @pl.when(c == 0)
    def _():
        agg_ref[...] = lax.dot_general(
            m_ref[...], xt_ref[...], (((1,), (0,)), ((), ())),
            preferred_element_type=jnp.float32)

    cdims = (((1,), (1,)), ((), ()))
    o_ref[...] = (
        lax.dot_general(agg_ref[...], wl_ref[...], cdims,
                        preferred_element_type=jnp.float32)
        + lax.dot_general(xt_ref[...], wr_ref[...], cdims,
                          preferred_element_type=jnp.float32)
        + jnp.broadcast_to(b_ref[...], (N_GENES, _GBC))
    )


def _gene_tail_body(xt_ref, agg_ref, wl_ref, wr_ref, b_ref, o_ref):
    cdims = (((1,), (1,)), ((), ()))
    o_ref[...] = (
        lax.dot_general(agg_ref[...], wl_ref[...], cdims,
                        preferred_element_type=jnp.float32)
        + lax.dot_general(xt_ref[...], wr_ref[...], cdims,
                          preferred_element_type=jnp.float32)
        + jnp.broadcast_to(b_ref[...], (N_GENES, _GTAIL))
    )


def _tc_gene_layer(xt, mn, wl, wr, b2d):
    main, agg = pl.pallas_call(
        _gene_layer_body,
        grid=(_GMAIN // _GBC,),
        in_specs=[
            pl.BlockSpec((N_GENES, N_CELLS), lambda c: (0, 0)),
            pl.BlockSpec((N_GENES, N_GENES), lambda c: (0, 0)),
            pl.BlockSpec((_GBC, N_CELLS), lambda c: (c, 0)),
            pl.BlockSpec((_GBC, N_CELLS), lambda c: (c, 0)),
            pl.BlockSpec((1, _GBC), lambda c: (0, c)),
        ],
        out_specs=[
            pl.BlockSpec((N_GENES, _GBC), lambda c: (0, c)),
            pl.BlockSpec((N_GENES, N_CELLS), lambda c: (0, 0)),
        ],
        out_shape=[
            jax.ShapeDtypeStruct((N_GENES, _GMAIN), jnp.float32),
            jax.ShapeDtypeStruct((N_GENES, N_CELLS), jnp.float32),
        ],
    )(xt, mn, wl, wr, b2d)
    tail = pl.pallas_call(
        _gene_tail_body,
        grid=(1,),
        in_specs=[
            pl.BlockSpec((N_GENES, N_CELLS), lambda c: (0, 0)),
            pl.BlockSpec((N_GENES, N_CELLS), lambda c: (0, 0)),
            pl.BlockSpec((_GTAIL, N_CELLS), lambda c: (_GMAIN // _GTAIL, 0)),
            pl.BlockSpec((_GTAIL, N_CELLS), lambda c: (_GMAIN // _GTAIL, 0)),
            pl.BlockSpec((1, _GTAIL), lambda c: (0, 0)),
        ],
        out_specs=pl.BlockSpec((N_GENES, _GTAIL), lambda c: (0, 0)),
        out_shape=jax.ShapeDtypeStruct((N_GENES, _GTAIL), jnp.float32),
    )(xt, agg, wl, wr, b2d[:, _GMAIN:])
    return jnp.concatenate([main, tail], axis=1)


def _pack_edges(idx, pad_vals):
    pad = _NW * _CH * _CK - E_CELL
    e80 = jnp.concatenate([idx, pad_vals[:pad]])
    e80 = e80.reshape(_NW, _CH, _CK)
    dummy = jnp.broadcast_to(
        pad_vals[: (_CHB - _CH) * _CK].reshape(1, _CHB - _CH, _CK),
        (_NW, _CHB - _CH, _CK))
    return jnp.concatenate([e80, dummy], axis=1)


def kernel(x, cell_graph, gene_graph, Wl_c, Wr_c, b_c, Wl_g, Wr_g, b_g):
    npd = _NW * _CH * _CK - E_CELL + (_CHB - _CH) * _CK
    k = jnp.arange(npd, dtype=jnp.int32)
    srcp = _pack_edges(cell_graph[0], (k * 79) % N_CELLS)
    dstp = _pack_edges(cell_graph[1], N_CELLS + (k % (_NPAD - N_CELLS)))
    b_c2 = b_c.reshape(1, N_GENES)
    b_g2 = b_g.reshape(1, N_CELLS)

    mn = _tc_gene_m(gene_graph)
    hist = _tc_cnt_hist(cell_graph[1].reshape(1, E_CELL))
    cnt_col = hist.reshape(80 * 128)[:N_CELLS].reshape(N_CELLS, 1)

    zeros = jnp.zeros((_NPAD, 128), jnp.float32)
    h = x
    for _ in range(2):
        xe = jnp.concatenate([h, jnp.zeros((1, N_GENES), jnp.float32)])
        sums = _make_sc_cell_agg()(xe, srcp, dstp, zeros)
        h = _tc_cell_update(sums, cnt_col, h, Wl_c, Wr_c, b_c2)

    ht = h.T
    for _ in range(2):
        ht = _tc_gene_layer(ht, mn, Wl_g, Wr_g, b_g2)
    return ht


# speedup vs baseline: 1.0022x; 1.0022x over previous
import functools

import jax
import jax.numpy as jnp
from jax import lax
from jax.experimental import pallas as pl
from jax.experimental.pallas import tpu as pltpu
from jax.experimental.pallas import tpu_sc as plsc

N_CELLS = 10000
N_GENES = 128
E_CELL = 320000
E_GENE = 2048

_NC = 2
_NS = 16
_NW = _NC * _NS
_CK = 128
_CH = 80
_CHB = 88
_NPAD = 10112
_RPS = _NPAD // _NS

@functools.lru_cache(maxsize=1)
def _make_sc_cell_agg():
    mesh = plsc.VectorSubcoreMesh(core_axis_name="c", subcore_axis_name="s")

    @functools.partial(
        pl.kernel,
        mesh=mesh,
        out_type=jax.ShapeDtypeStruct((_NC, _NPAD, 128), jnp.float32),
        scratch_types=[
            pltpu.VMEM((_CHB, _CK), jnp.int32),
            pltpu.VMEM((_CHB, _CK), jnp.int32),
            pltpu.VMEM((_CK, 128), jnp.float32),
            pltpu.VMEM_SHARED((_NPAD, 128), jnp.float32),
            pltpu.SemaphoreType.DMA,
        ],
    )
    def _sc_cell_agg(xe_hbm, srcp_hbm, dstp_hbm, zeros_hbm, sum_out,
                     src_v, dst_v, buf_a, ssum, sem_a):
        cid = lax.axis_index("c")
        sid = lax.axis_index("s")
        w = cid * _NS + sid
        sl = pl.ds(sid * _RPS, _RPS)
        with jax.named_scope("agg_stage"):
            pltpu.sync_copy(zeros_hbm.at[sl], ssum.at[sl])
            pltpu.sync_copy(srcp_hbm.at[w], src_v)
            pltpu.sync_copy(dstp_hbm.at[w], dst_v)
        with jax.named_scope("agg_bar1"):
            plsc.subcore_barrier()

        def chunk(ci, carry):
            pltpu.async_copy(xe_hbm.at[src_v.at[ci]], buf_a, sem_a).wait()
            pltpu.sync_copy(buf_a, ssum.at[dst_v.at[ci]], add=True)
            return carry

        with jax.named_scope("agg_chunks"):
            lax.fori_loop(0, _CH, chunk, 0)
        with jax.named_scope("agg_bar2"):
            plsc.subcore_barrier()
        with jax.named_scope("agg_writeout"):
            pltpu.sync_copy(ssum.at[sl], sum_out.at[cid].at[sl])

    return _sc_cell_agg


_CNT_BLK = 12800


def _cnt_hist_body(dst_ref, o_ref):
    i = pl.program_id(0)

    @pl.when(i == 0)
    def _():
        o_ref[...] = jnp.zeros((80, 128), jnp.float32)

    d = dst_ref[...]
    hi = d >> 7
    lo = d & 127
    ioh = lax.broadcasted_iota(jnp.int32, (80, _CNT_BLK), 0)
    iol = lax.broadcasted_iota(jnp.int32, (128, _CNT_BLK), 0)
    dh = (ioh == hi).astype(jnp.float32)
    dl = (iol == lo).astype(jnp.float32)
    o_ref[...] += lax.dot_general(dh, dl, (((1,), (1,)), ((), ())),
                                  preferred_element_type=jnp.float32)


def _tc_cnt_hist(dst2):
    return pl.pallas_call(
        _cnt_hist_body,
        grid=(E_CELL // _CNT_BLK,),
        in_specs=[pl.BlockSpec((1, _CNT_BLK), lambda i: (0, i))],
        out_specs=pl.BlockSpec((80, 128), lambda i: (0, 0)),
        out_shape=jax.ShapeDtypeStruct((80, 128), jnp.float32),
    )(dst2)
def _cell_update_body(sum_ref, cnt_ref, x_ref, wl_ref, wr_ref, b_ref, o_ref):
    s = sum_ref[0] + sum_ref[1]
    r = 1.0 / jnp.clip(cnt_ref[...], 1.0, None)
    agg = s * r
    cdims = (((1,), (1,)), ((), ()))
    o_ref[...] = (
        lax.dot_general(agg, wl_ref[...], cdims,
                        preferred_element_type=jnp.float32)
        + lax.dot_general(x_ref[...], wr_ref[...], cdims,
                          preferred_element_type=jnp.float32)
        + b_ref[...]
    )


def _tc_cell_update(sum_parts, cnt_col, x, wl, wr, b2d):
    br = 1000
    grid = (N_CELLS // br,)
    return pl.pallas_call(
        _cell_update_body,
        grid=grid,
        in_specs=[
            pl.BlockSpec((_NC, br, 128), lambda i: (0, i, 0)),
            pl.BlockSpec((br, 1), lambda i: (i, 0)),
            pl.BlockSpec((br, 128), lambda i: (i, 0)),
            pl.BlockSpec((128, 128), lambda i: (0, 0)),
            pl.BlockSpec((128, 128), lambda i: (0, 0)),
            pl.BlockSpec((1, 128), lambda i: (0, 0)),
        ],
        out_specs=pl.BlockSpec((br, 128), lambda i: (i, 0)),
        out_shape=jax.ShapeDtypeStruct((N_CELLS, 128), jnp.float32),
    )(sum_parts, cnt_col, x, wl, wr, b2d)


def _gene_m_body(gg_ref, o_ref):
    gg = gg_ref[...]
    src = gg[0:1, :]
    dst = gg[1:2, :]
    ids = lax.broadcasted_iota(jnp.int32, (N_GENES, E_GENE), 0)
    dhot = (ids == dst).astype(jnp.float32)
    shot = (ids == src).astype(jnp.float32)
    m_raw = lax.dot_general(dhot, shot, (((1,), (1,)), ((), ())),
                            preferred_element_type=jnp.float32)
    cnt = jnp.sum(dhot, axis=1, keepdims=True)
    o_ref[...] = m_raw * (1.0 / jnp.clip(cnt, 1.0, None))


def _tc_gene_m(gene_graph):
    return pl.pallas_call(
        _gene_m_body,
        out_shape=jax.ShapeDtypeStruct((N_GENES, N_GENES), jnp.float32),
    )(gene_graph)


_GBC = 256
_GMAIN = (N_CELLS // _GBC) * _GBC
_GTAIL = N_CELLS - _GMAIN


def _gene_layer_body(xt_ref, m_ref, wl_ref, wr_ref, b_ref, o_ref, agg_ref):
    c = pl.program_id(0)

    @pl.when(c == 0)
    def _():
        agg_ref[...] = lax.dot_general(
            m_ref[...], xt_ref[...], (((1,), (0,)), ((), ())),
            preferred_element_type=jnp.float32)

    cdims = (((1,), (1,)), ((), ()))
    o_ref[...] = (
        lax.dot_general(agg_ref[...], wl_ref[...], cdims,
                        preferred_element_type=jnp.float32)
        + lax.dot_general(xt_ref[...], wr_ref[...], cdims,
                          preferred_element_type=jnp.float32)
        + jnp.broadcast_to(b_ref[...], (N_GENES, _GBC))
    )


def _gene_tail_body(xt_ref, agg_ref, wl_ref, wr_ref, b_ref, o_ref):
    cdims = (((1,), (1,)), ((), ()))
    o_ref[...] = (
        lax.dot_general(agg_ref[...], wl_ref[...], cdims,
                        preferred_element_type=jnp.float32)
        + lax.dot_general(xt_ref[...], wr_ref[...], cdims,
                          preferred_element_type=jnp.float32)
        + jnp.broadcast_to(b_ref[...], (N_GENES, _GTAIL))
    )


def _tc_gene_layer(xt, mn, wl, wr, b2d):
    main, agg = pl.pallas_call(
        _gene_layer_body,
        grid=(_GMAIN // _GBC,),
        in_specs=[
            pl.BlockSpec((N_GENES, N_CELLS), lambda c: (0, 0)),
            pl.BlockSpec((N_GENES, N_GENES), lambda c: (0, 0)),
            pl.BlockSpec((_GBC, N_CELLS), lambda c: (c, 0)),
            pl.BlockSpec((_GBC, N_CELLS), lambda c: (c, 0)),
            pl.BlockSpec((1, _GBC), lambda c: (0, c)),
        ],
        out_specs=[
            pl.BlockSpec((N_GENES, _GBC), lambda c: (0, c)),
            pl.BlockSpec((N_GENES, N_CELLS), lambda c: (0, 0)),
        ],
        out_shape=[
            jax.ShapeDtypeStruct((N_GENES, _GMAIN), jnp.float32),
            jax.ShapeDtypeStruct((N_GENES, N_CELLS), jnp.float32),
        ],
    )(xt, mn, wl, wr, b2d)
    tail = pl.pallas_call(
        _gene_tail_body,
        grid=(1,),
        in_specs=[
            pl.BlockSpec((N_GENES, N_CELLS), lambda c: (0, 0)),
            pl.BlockSpec((N_GENES, N_CELLS), lambda c: (0, 0)),
            pl.BlockSpec((_GTAIL, N_CELLS), lambda c: (_GMAIN // _GTAIL, 0)),
            pl.BlockSpec((_GTAIL, N_CELLS), lambda c: (_GMAIN // _GTAIL, 0)),
            pl.BlockSpec((1, _GTAIL), lambda c: (0, 0)),
        ],
        out_specs=pl.BlockSpec((N_GENES, _GTAIL), lambda c: (0, 0)),
        out_shape=jax.ShapeDtypeStruct((N_GENES, _GTAIL), jnp.float32),
    )(xt, agg, wl, wr, b2d[:, _GMAIN:])
    return jnp.concatenate([main, tail], axis=1)


def _pack_edges(idx, pad_vals):
    pad = _NW * _CH * _CK - E_CELL
    e80 = jnp.concatenate([idx, pad_vals[:pad]])
    e80 = e80.reshape(_NW, _CH, _CK)
    dummy = jnp.broadcast_to(
        pad_vals[: (_CHB - _CH) * _CK].reshape(1, _CHB - _CH, _CK),
        (_NW, _CHB - _CH, _CK))
    return jnp.concatenate([e80, dummy], axis=1)


def kernel(x, cell_graph, gene_graph, Wl_c, Wr_c, b_c, Wl_g, Wr_g, b_g):
    npd = _NW * _CH * _CK - E_CELL + (_CHB - _CH) * _CK
    k = jnp.arange(npd, dtype=jnp.int32)
    srcp = _pack_edges(cell_graph[0], (k * 79) % N_CELLS)
    dstp = _pack_edges(cell_graph[1], N_CELLS + (k % (_NPAD - N_CELLS)))
    b_c2 = b_c.reshape(1, N_GENES)
    b_g2 = b_g.reshape(1, N_CELLS)

    mn = _tc_gene_m(gene_graph)
    hist = _tc_cnt_hist(cell_graph[1].reshape(1, E_CELL))
    cnt_col = hist.reshape(80 * 128)[:N_CELLS].reshape(N_CELLS, 1)

    zeros = jnp.zeros((_NPAD, 128), jnp.float32)
    h = x
    for _ in range(2):
        xe = jnp.concatenate([h, jnp.zeros((1, N_GENES), jnp.float32)])
        sums = _make_sc_cell_agg()(xe, srcp, dstp, zeros)
        h = _tc_cell_update(sums, cnt_col, h, Wl_c, Wr_c, b_c2)

    ht = h.T
    for _ in range(2):
        ht = _tc_gene_layer(ht, mn, Wl_g, Wr_g, b_g2)
    return ht
